# Initial kernel scaffold; baseline (speedup 1.0000x reference)
import jax
import jax.numpy as jnp
from jax.experimental import pallas as pl


def kernel(input_ids, token_type_ids, word_table, ax1, ax2, tt_table, gamma, beta):
    raise NotImplementedError("write your pallas kernel here")


# speedup vs baseline: 1.6763x; 1.6763x over previous
import functools

import jax
import jax.numpy as jnp
from jax import lax
from jax.experimental import pallas as pl
from jax.experimental.pallas import tpu as pltpu
from jax.experimental.pallas import tpu_sc as plsc

HID = 1024
F = 32
SEQ = F * F
B = 16
NT = B * SEQ
EPS = 1e-12

NC = 2
NS = 16
NW = NC * NS
ROWS_PER_W = NT // NW
K = 32
NG = ROWS_PER_W // K

ROWS_BLK = 256
GRID = NT // ROWS_BLK
A1B = ROWS_BLK // F


def _sc_gather(table, ids3d):
    mesh = plsc.VectorSubcoreMesh(core_axis_name="c", subcore_axis_name="s")

    @functools.partial(
        pl.kernel,
        mesh=mesh,
        out_type=jax.ShapeDtypeStruct((NT, HID), jnp.float32),
        scratch_types=[
            pltpu.VMEM((NG, K), jnp.int32),
            pltpu.VMEM((K, HID), jnp.float32),
            pltpu.VMEM((K, HID), jnp.float32),
            pltpu.SemaphoreType.DMA,
            pltpu.SemaphoreType.DMA,
            pltpu.SemaphoreType.DMA,
            pltpu.SemaphoreType.DMA,
        ],
    )
    def body(table_hbm, ids_hbm, out_hbm, idx_v, buf0, buf1, g0, g1, w0, w1):
        wid = lax.axis_index("s") * NC + lax.axis_index("c")
        base = wid * ROWS_PER_W
        pltpu.sync_copy(ids_hbm.at[wid], idx_v)

        bufs = [buf0, buf1]
        gsem = [g0, g1]
        wsem = [w0, w1]
        gcp = [None, None]
        wcp = [None, None]

        gcp[0] = pltpu.async_copy(table_hbm.at[idx_v.at[0]], bufs[0], gsem[0])
        for g in range(NG):
            bi = g & 1
            ni = (g + 1) & 1
            if g + 1 < NG:
                if wcp[ni] is not None:
                    wcp[ni].wait()
                gcp[ni] = pltpu.async_copy(
                    table_hbm.at[idx_v.at[g + 1]], bufs[ni], gsem[ni]
                )
            gcp[bi].wait()
            wcp[bi] = pltpu.async_copy(
                bufs[bi], out_hbm.at[pl.ds(base + g * K, K)], wsem[bi]
            )
        wcp[0].wait()
        wcp[1].wait()

    return body(table, ids3d)


def _tc_epilogue(emb, ax1, ax2, ttpad, dtt, gamma, beta):
    def body(emb_ref, ax1_ref, ax2_ref, tt_ref, dtt_ref, g_ref, b_ref, o_ref):
        x = emb_ref[...].reshape(A1B, F, HID)
        x = x + ax1_ref[...][:, None, :] + ax2_ref[...][None, :, :]
        ttf = jnp.sum(tt_ref[...], axis=1, keepdims=True)
        x = x + (ttf * dtt_ref[...]).reshape(A1B, F, HID)
        mean = jnp.mean(x, axis=2, keepdims=True)
        var = jnp.mean(x * x, axis=2, keepdims=True) - mean * mean
        inv = lax.rsqrt(var + EPS)
        y = (x - mean) * inv * g_ref[...][None, :, :] + b_ref[...][None, :, :]
        o_ref[...] = y.reshape(ROWS_BLK, HID)

    return pl.pallas_call(
        body,
        grid=(GRID,),
        in_specs=[
            pl.BlockSpec((ROWS_BLK, HID), lambda t: (t, 0)),
            pl.BlockSpec((A1B, HID), lambda t: (t % (F // A1B), 0)),
            pl.BlockSpec((F, HID), lambda t: (0, 0)),
            pl.BlockSpec((ROWS_BLK, 8), lambda t: (t, 0)),
            pl.BlockSpec((1, HID), lambda t: (0, 0)),
            pl.BlockSpec((1, HID), lambda t: (0, 0)),
            pl.BlockSpec((1, HID), lambda t: (0, 0)),
        ],
        out_specs=pl.BlockSpec((ROWS_BLK, HID), lambda t: (t, 0)),
        out_shape=jax.ShapeDtypeStruct((NT, HID), jnp.float32),
    )(emb, ax1, ax2, ttpad, dtt, gamma, beta)


def kernel(input_ids, token_type_ids, word_table, ax1, ax2, tt_table, gamma,
           beta):
    ids3d = input_ids.astype(jnp.int32).reshape(NW, NG, K)
    emb = _sc_gather(word_table, ids3d)

    ttf = token_type_ids.astype(jnp.float32).reshape(NT, 1)
    ttpad = jnp.pad(ttf, ((0, 0), (0, 7)))
    ax1p = ax1 + tt_table[0:1]
    dtt = tt_table[1:2] - tt_table[0:1]
    out = _tc_epilogue(emb, ax1p, ax2, ttpad, dtt,
                       gamma.reshape(1, HID), beta.reshape(1, HID))
    return out.reshape(B, SEQ, HID)
